# baseline (speedup 1.0000x reference)
import functools

import jax
import jax.numpy as jnp
from jax import lax
from jax.experimental import pallas as pl
from jax.experimental.pallas import tpu as pltpu
from jax.experimental.pallas import tpu_sc as plsc

B = 10000
C = 4
M = 8
D = 128
NW = 32
CHUNK = 320
BLK = 8
STAGE_CHUNK = 80
SELF_CHUNK = 40
RES_RATE = 0.9


def _sc_gather_body(nodes_hbm, neigh_hbm, fa_hbm, ff_hbm,
                    out_n, out_a, out_f,
                    idx_v, idx2_v, nodes_v, rows0_v, rows1_v,
                    sums0_v, sums1_v, self_v,
                    gsem0, gsem1, wsem0, wsem1, ssem):
    rows_b = (rows0_v, rows1_v)
    sums_b = (sums0_v, sums1_v)
    wid = lax.axis_index("s") * 2 + lax.axis_index("c")
    start = pl.multiple_of(wid * CHUNK, 8)
    cnt = jnp.minimum(CHUNK, B - start)
    nstage = cnt // STAGE_CHUNK
    nself = cnt // SELF_CHUNK

    def stage_body(k, carry):
        off = pl.multiple_of(start + k * STAGE_CHUNK, 8)
        pltpu.sync_copy(neigh_hbm.at[pl.ds(off, STAGE_CHUNK)], idx_v)

        def repack(q, carry2):
            dst = k * (STAGE_CHUNK * C * M // 128) + q
            for j in range(8):
                idx2_v[dst, pl.ds(j * 16, 16)] = (
                    idx_v[4 * q + j // 2, pl.ds((j % 2) * 16, 16)])
            return carry2

        lax.fori_loop(0, STAGE_CHUNK * C * M // 128, repack, 0)
        return carry

    lax.fori_loop(0, nstage, stage_body, 0)

    def self_body(k, carry):
        noff = pl.multiple_of(start + k * SELF_CHUNK, 8)
        voff = pl.multiple_of(k * SELF_CHUNK, 8)
        pltpu.sync_copy(nodes_hbm.at[pl.ds(noff, SELF_CHUNK)],
                        nodes_v.at[pl.ds(voff, SELF_CHUNK)])
        idxs = nodes_v.at[pl.ds(voff, SELF_CHUNK)]
        ca = pltpu.async_copy(fa_hbm.at[idxs], self_v.at[0], ssem)
        cf = pltpu.async_copy(ff_hbm.at[idxs], self_v.at[1], ssem)
        ca.wait()
        cf.wait()
        pltpu.sync_copy(self_v.at[0], out_a.at[pl.ds(noff, SELF_CHUNK)])
        pltpu.sync_copy(self_v.at[1], out_f.at[pl.ds(noff, SELF_CHUNK)])
        return carry

    lax.fori_loop(0, nself, self_body, 0)

    nblk = cnt // BLK
    npair = nblk // 2

    gsems = (gsem0, gsem1)
    wsems = (wsem0, wsem1)

    def issue_gather(g, sb):
        pltpu.async_copy(fa_hbm.at[idx2_v.at[2 * g]],
                         rows_b[sb].at[pl.ds(0, 128)], gsems[sb])
        pltpu.async_copy(fa_hbm.at[idx2_v.at[2 * g + 1]],
                         rows_b[sb].at[pl.ds(128, 128)], gsems[sb])

    def wait_gather(sb):
        pltpu.make_async_copy(fa_hbm.at[idx2_v.at[0]],
                              rows_b[sb].at[pl.ds(0, 128)], gsems[sb]).wait()
        pltpu.make_async_copy(fa_hbm.at[idx2_v.at[0]],
                              rows_b[sb].at[pl.ds(128, 128)], gsems[sb]).wait()

    def compute(sb):
        def obody(q, carry):
            for u in range(4):
                o = 4 * q + u
                src = o * M
                for k in range(D // 16):
                    acc = rows_b[sb][src, pl.ds(k * 16, 16)]
                    for m in range(1, M):
                        acc = acc + rows_b[sb][src + m, pl.ds(k * 16, 16)]
                    sums_b[sb][o // C, o % C, pl.ds(k * 16, 16)] = acc
            return carry

        lax.fori_loop(0, C * BLK // 4, obody, 0)

    def issue_writes(g, sb):
        nbase = start + BLK * g
        pltpu.async_copy(sums_b[sb], out_n.at[pl.ds(nbase, BLK)], wsems[sb])

    def wait_writes(sb):
        pltpu.make_async_copy(sums_b[sb], out_n.at[pl.ds(0, BLK)],
                              wsems[sb]).wait()

    issue_gather(0, 0)

    def pair_body(p, carry):
        g0 = 2 * p
        issue_gather(g0 + 1, 1)
        wait_gather(0)

        @pl.when(p > 0)
        def _():
            wait_writes(0)

        compute(0)
        issue_writes(g0, 0)

        @pl.when(g0 + 2 < nblk)
        def _():
            issue_gather(g0 + 2, 0)

        wait_gather(1)

        @pl.when(p > 0)
        def _():
            wait_writes(1)

        compute(1)
        issue_writes(g0 + 1, 1)
        return carry

    lax.fori_loop(0, npair, pair_body, 0)
    wait_writes(0)
    wait_writes(1)


def _sc_gather(nodes, neigh2, feat_agg, feat_ff):
    mesh = plsc.VectorSubcoreMesh(core_axis_name="c", subcore_axis_name="s")
    f = functools.partial(
        pl.kernel,
        mesh=mesh,
        out_type=(
            jax.ShapeDtypeStruct((B, C, D), jnp.float32),
            jax.ShapeDtypeStruct((B, D), jnp.float32),
            jax.ShapeDtypeStruct((B, D), jnp.float32),
        ),
        scratch_types=[
            pltpu.VMEM((STAGE_CHUNK, C * M), jnp.int32),
            pltpu.VMEM((CHUNK * C * M // 128, 128), jnp.int32),
            pltpu.VMEM((CHUNK,), jnp.int32),
            pltpu.VMEM((2 * 128, D), jnp.float32),
            pltpu.VMEM((2 * 128, D), jnp.float32),
            pltpu.VMEM((BLK, C, D), jnp.float32),
            pltpu.VMEM((BLK, C, D), jnp.float32),
            pltpu.VMEM((2, SELF_CHUNK, D), jnp.float32),
            pltpu.SemaphoreType.DMA,
            pltpu.SemaphoreType.DMA,
            pltpu.SemaphoreType.DMA,
            pltpu.SemaphoreType.DMA,
            pltpu.SemaphoreType.DMA,
        ],
    )(_sc_gather_body)
    return f(nodes, neigh2, feat_agg, feat_ff)


def _tc_dense_body(n_ref, a_ref, f_ref,
                   wagg_ref, wff_ref, wk_ref, wq_ref, mu_ref,
                   outa_ref, outf_ref):
    A = a_ref[...]
    F = f_ref[...]
    Wagg = wagg_ref[...]
    mu = mu_ref[...]
    mu_a = mu[0:1, :]
    mu_n = mu[1:2, :]

    dot = functools.partial(jnp.dot, preferred_element_type=jnp.float32)
    agg_v = dot(A, Wagg)
    ff_v = dot(F, wff_ref[...])
    Ka = dot(A, wk_ref[...])
    Kf = dot(F, wk_ref[...])
    Qa = dot(A, wq_ref[...])
    Qf = dot(F, wq_ref[...])

    a2 = jnp.sum(agg_v * agg_v, axis=1, keepdims=True)
    da = jnp.sum(agg_v * mu_a, axis=1, keepdims=True)

    neigh_aggs = []
    logits = []
    for c in range(C):
        NA = dot(n_ref[:, c, :] * (1.0 / M), Wagg)
        n2 = jnp.sum(NA * NA, axis=1, keepdims=True)
        dn = jnp.sum(NA * mu_n, axis=1, keepdims=True)
        norm = jnp.maximum(jnp.sqrt(a2 + n2), 1e-12)
        neigh_aggs.append(NA)
        logits.append((da + dn) / norm)

    mx = jnp.maximum(jnp.maximum(logits[0], logits[1]),
                     jnp.maximum(logits[2], logits[3]))
    es = [jnp.exp(l - mx) for l in logits]
    inv_z = 1.0 / (es[0] + es[1] + es[2] + es[3])
    comb = (es[0] * neigh_aggs[0] + es[1] * neigh_aggs[1]
            + es[2] * neigh_aggs[2] + es[3] * neigh_aggs[3]) * inv_z
    agg_v = (agg_v + comb) * 0.5

    inv_d = 1.0 / D
    s00 = jnp.sum(Ka * Qa, axis=1, keepdims=True) * inv_d
    s01 = jnp.sum(Ka * Qf, axis=1, keepdims=True) * inv_d
    s10 = jnp.sum(Kf * Qa, axis=1, keepdims=True) * inv_d
    s11 = jnp.sum(Kf * Qf, axis=1, keepdims=True) * inv_d

    m0 = jnp.maximum(s00, s01)
    e00 = jnp.exp(s00 - m0)
    e01 = jnp.exp(s01 - m0)
    iz0 = 1.0 / (e00 + e01)
    m1 = jnp.maximum(s10, s11)
    e10 = jnp.exp(s10 - m1)
    e11 = jnp.exp(s11 - m1)
    iz1 = 1.0 / (e10 + e11)

    new_a = (e00 * agg_v + e01 * ff_v) * iz0
    new_f = (e10 * agg_v + e11 * ff_v) * iz1

    xa = RES_RATE * agg_v + (1.0 - RES_RATE) * new_a
    xf = RES_RATE * ff_v + (1.0 - RES_RATE) * new_f
    outa_ref[...] = jnp.where(xa > 0, xa, jnp.exp(jnp.minimum(xa, 0.0)) - 1.0)
    outf_ref[...] = jnp.where(xf > 0, xf, jnp.exp(jnp.minimum(xf, 0.0)) - 1.0)


def _tc_dense(neigh3, selfa, selff, W_agg_v, W_ff_v, W_k, W_q, mu2):
    R = 1000
    grid = (B // R,)
    nspec = [pl.BlockSpec((R, C, D), lambda i: (i, 0, 0))]
    rspec = pl.BlockSpec((R, D), lambda i: (i, 0))
    wspec = pl.BlockSpec((D, D), lambda i: (0, 0))
    muspec = pl.BlockSpec((2, D), lambda i: (0, 0))
    return pl.pallas_call(
        _tc_dense_body,
        grid=grid,
        in_specs=nspec + [rspec, rspec, wspec, wspec, wspec, wspec, muspec],
        out_specs=[rspec, rspec],
        out_shape=[jax.ShapeDtypeStruct((B, D), jnp.float32),
                   jax.ShapeDtypeStruct((B, D), jnp.float32)],
    )(neigh3, selfa, selff,
      W_agg_v, W_ff_v, W_k, W_q, mu2)


def kernel(nodes, neigh_idx, feat_agg, feat_ff, W_agg_v, W_ff_v, W_k, W_q, W_mu):
    neigh3, selfa, selff = _sc_gather(nodes, neigh_idx.reshape(B, C * M),
                                      feat_agg, feat_ff)
    mu2 = W_mu.reshape(2, D)
    out_agg, out_ff = _tc_dense(neigh3, selfa, selff,
                                W_agg_v, W_ff_v, W_k, W_q, mu2)
    return (out_agg, out_ff)


# speedup vs baseline: 1.3121x; 1.3121x over previous
import functools

import jax
import jax.numpy as jnp
from jax import lax
from jax.experimental import pallas as pl
from jax.experimental.pallas import tpu as pltpu
from jax.experimental.pallas import tpu_sc as plsc

B = 10000
C = 4
M = 8
D = 128
NW = 32
CHUNK = 320
BLK = 8
STAGE_CHUNK = 80
SELF_CHUNK = 40
RES_RATE = 0.9


def _sc_gather_body(nodes_hbm, neigh_hbm, fa_hbm, ff_hbm,
                    out_n, out_a, out_f,
                    idx_v, idx2_v, nodes_v, rows0_v, rows1_v,
                    sums0_v, sums1_v, self_v,
                    gsem0, gsem1, wsem0, wsem1, ssem):
    rows_b = (rows0_v, rows1_v)
    sums_b = (sums0_v, sums1_v)
    wid = lax.axis_index("s") * 2 + lax.axis_index("c")
    start = pl.multiple_of(wid * CHUNK, 8)
    cnt = jnp.minimum(CHUNK, B - start)
    nstage = cnt // STAGE_CHUNK
    nself = cnt // SELF_CHUNK

    def stage_body(k, carry):
        off = pl.multiple_of(start + k * STAGE_CHUNK, 8)
        pltpu.sync_copy(neigh_hbm.at[pl.ds(off, STAGE_CHUNK)], idx_v)

        def repack(q, carry2):
            dst = k * (STAGE_CHUNK * C * M // 128) + q
            for j in range(8):
                idx2_v[dst, pl.ds(j * 16, 16)] = (
                    idx_v[4 * q + j // 2, pl.ds((j % 2) * 16, 16)])
            return carry2

        lax.fori_loop(0, STAGE_CHUNK * C * M // 128, repack, 0)
        return carry

    lax.fori_loop(0, nstage, stage_body, 0)

    def self_body(k, carry):
        noff = pl.multiple_of(start + k * SELF_CHUNK, 8)
        voff = pl.multiple_of(k * SELF_CHUNK, 8)
        pltpu.sync_copy(nodes_hbm.at[pl.ds(noff, SELF_CHUNK)],
                        nodes_v.at[pl.ds(voff, SELF_CHUNK)])
        idxs = nodes_v.at[pl.ds(voff, SELF_CHUNK)]
        ca = pltpu.async_copy(fa_hbm.at[idxs], self_v.at[0], ssem)
        cf = pltpu.async_copy(ff_hbm.at[idxs], self_v.at[1], ssem)
        ca.wait()
        cf.wait()
        pltpu.sync_copy(self_v.at[0], out_a.at[pl.ds(noff, SELF_CHUNK)])
        pltpu.sync_copy(self_v.at[1], out_f.at[pl.ds(noff, SELF_CHUNK)])
        return carry

    lax.fori_loop(0, nself, self_body, 0)

    nblk = cnt // BLK
    npair = nblk // 2

    gsems = (gsem0, gsem1)
    wsems = (wsem0, wsem1)

    def issue_gather(g, sb):
        pltpu.async_copy(fa_hbm.at[idx2_v.at[2 * g]],
                         rows_b[sb].at[pl.ds(0, 128)], gsems[sb])
        pltpu.async_copy(fa_hbm.at[idx2_v.at[2 * g + 1]],
                         rows_b[sb].at[pl.ds(128, 128)], gsems[sb])

    def wait_gather(sb):
        pltpu.make_async_copy(fa_hbm.at[idx2_v.at[0]],
                              rows_b[sb].at[pl.ds(0, 128)], gsems[sb]).wait()
        pltpu.make_async_copy(fa_hbm.at[idx2_v.at[0]],
                              rows_b[sb].at[pl.ds(128, 128)], gsems[sb]).wait()

    def compute(sb):
        @plsc.parallel_loop(0, C * BLK, 1, unroll=4)
        def obody(o):
            src = o * M
            for k in range(D // 16):
                r = [rows_b[sb][src + m, pl.ds(k * 16, 16)] for m in range(M)]
                s0 = r[0] + r[1]
                s1 = r[2] + r[3]
                s2 = r[4] + r[5]
                s3 = r[6] + r[7]
                sums_b[sb][o // C, o % C, pl.ds(k * 16, 16)] = (s0 + s1) + (s2 + s3)

    def issue_writes(g, sb):
        nbase = start + BLK * g
        pltpu.async_copy(sums_b[sb], out_n.at[pl.ds(nbase, BLK)], wsems[sb])

    def wait_writes(sb):
        pltpu.make_async_copy(sums_b[sb], out_n.at[pl.ds(0, BLK)],
                              wsems[sb]).wait()

    issue_gather(0, 0)

    def pair_body(p, carry):
        g0 = 2 * p
        issue_gather(g0 + 1, 1)
        wait_gather(0)

        @pl.when(p > 0)
        def _():
            wait_writes(0)

        compute(0)
        issue_writes(g0, 0)

        @pl.when(g0 + 2 < nblk)
        def _():
            issue_gather(g0 + 2, 0)

        wait_gather(1)

        @pl.when(p > 0)
        def _():
            wait_writes(1)

        compute(1)
        issue_writes(g0 + 1, 1)
        return carry

    lax.fori_loop(0, npair, pair_body, 0)
    wait_writes(0)
    wait_writes(1)


def _sc_gather(nodes, neigh2, feat_agg, feat_ff):
    mesh = plsc.VectorSubcoreMesh(core_axis_name="c", subcore_axis_name="s")
    f = functools.partial(
        pl.kernel,
        mesh=mesh,
        out_type=(
            jax.ShapeDtypeStruct((B, C, D), jnp.float32),
            jax.ShapeDtypeStruct((B, D), jnp.float32),
            jax.ShapeDtypeStruct((B, D), jnp.float32),
        ),
        scratch_types=[
            pltpu.VMEM((STAGE_CHUNK, C * M), jnp.int32),
            pltpu.VMEM((CHUNK * C * M // 128, 128), jnp.int32),
            pltpu.VMEM((CHUNK,), jnp.int32),
            pltpu.VMEM((2 * 128, D), jnp.float32),
            pltpu.VMEM((2 * 128, D), jnp.float32),
            pltpu.VMEM((BLK, C, D), jnp.float32),
            pltpu.VMEM((BLK, C, D), jnp.float32),
            pltpu.VMEM((2, SELF_CHUNK, D), jnp.float32),
            pltpu.SemaphoreType.DMA,
            pltpu.SemaphoreType.DMA,
            pltpu.SemaphoreType.DMA,
            pltpu.SemaphoreType.DMA,
            pltpu.SemaphoreType.DMA,
        ],
    )(_sc_gather_body)
    return f(nodes, neigh2, feat_agg, feat_ff)


def _tc_dense_body(n_ref, a_ref, f_ref,
                   wagg_ref, wff_ref, wk_ref, wq_ref, mu_ref,
                   outa_ref, outf_ref):
    A = a_ref[...]
    F = f_ref[...]
    Wagg = wagg_ref[...]
    mu = mu_ref[...]
    mu_a = mu[0:1, :]
    mu_n = mu[1:2, :]

    dot = functools.partial(jnp.dot, preferred_element_type=jnp.float32)
    agg_v = dot(A, Wagg)
    ff_v = dot(F, wff_ref[...])
    Ka = dot(A, wk_ref[...])
    Kf = dot(F, wk_ref[...])
    Qa = dot(A, wq_ref[...])
    Qf = dot(F, wq_ref[...])

    a2 = jnp.sum(agg_v * agg_v, axis=1, keepdims=True)
    da = jnp.sum(agg_v * mu_a, axis=1, keepdims=True)

    neigh_aggs = []
    logits = []
    for c in range(C):
        NA = dot(n_ref[:, c, :] * (1.0 / M), Wagg)
        n2 = jnp.sum(NA * NA, axis=1, keepdims=True)
        dn = jnp.sum(NA * mu_n, axis=1, keepdims=True)
        norm = jnp.maximum(jnp.sqrt(a2 + n2), 1e-12)
        neigh_aggs.append(NA)
        logits.append((da + dn) / norm)

    mx = jnp.maximum(jnp.maximum(logits[0], logits[1]),
                     jnp.maximum(logits[2], logits[3]))
    es = [jnp.exp(l - mx) for l in logits]
    inv_z = 1.0 / (es[0] + es[1] + es[2] + es[3])
    comb = (es[0] * neigh_aggs[0] + es[1] * neigh_aggs[1]
            + es[2] * neigh_aggs[2] + es[3] * neigh_aggs[3]) * inv_z
    agg_v = (agg_v + comb) * 0.5

    inv_d = 1.0 / D
    s00 = jnp.sum(Ka * Qa, axis=1, keepdims=True) * inv_d
    s01 = jnp.sum(Ka * Qf, axis=1, keepdims=True) * inv_d
    s10 = jnp.sum(Kf * Qa, axis=1, keepdims=True) * inv_d
    s11 = jnp.sum(Kf * Qf, axis=1, keepdims=True) * inv_d

    m0 = jnp.maximum(s00, s01)
    e00 = jnp.exp(s00 - m0)
    e01 = jnp.exp(s01 - m0)
    iz0 = 1.0 / (e00 + e01)
    m1 = jnp.maximum(s10, s11)
    e10 = jnp.exp(s10 - m1)
    e11 = jnp.exp(s11 - m1)
    iz1 = 1.0 / (e10 + e11)

    new_a = (e00 * agg_v + e01 * ff_v) * iz0
    new_f = (e10 * agg_v + e11 * ff_v) * iz1

    xa = RES_RATE * agg_v + (1.0 - RES_RATE) * new_a
    xf = RES_RATE * ff_v + (1.0 - RES_RATE) * new_f
    outa_ref[...] = jnp.where(xa > 0, xa, jnp.exp(jnp.minimum(xa, 0.0)) - 1.0)
    outf_ref[...] = jnp.where(xf > 0, xf, jnp.exp(jnp.minimum(xf, 0.0)) - 1.0)


def _tc_dense(neigh3, selfa, selff, W_agg_v, W_ff_v, W_k, W_q, mu2):
    R = 1000
    grid = (B // R,)
    nspec = [pl.BlockSpec((R, C, D), lambda i: (i, 0, 0))]
    rspec = pl.BlockSpec((R, D), lambda i: (i, 0))
    wspec = pl.BlockSpec((D, D), lambda i: (0, 0))
    muspec = pl.BlockSpec((2, D), lambda i: (0, 0))
    return pl.pallas_call(
        _tc_dense_body,
        grid=grid,
        in_specs=nspec + [rspec, rspec, wspec, wspec, wspec, wspec, muspec],
        out_specs=[rspec, rspec],
        out_shape=[jax.ShapeDtypeStruct((B, D), jnp.float32),
                   jax.ShapeDtypeStruct((B, D), jnp.float32)],
    )(neigh3, selfa, selff,
      W_agg_v, W_ff_v, W_k, W_q, mu2)


def kernel(nodes, neigh_idx, feat_agg, feat_ff, W_agg_v, W_ff_v, W_k, W_q, W_mu):
    neigh3, selfa, selff = _sc_gather(nodes, neigh_idx.reshape(B, C * M),
                                      feat_agg, feat_ff)
    mu2 = W_mu.reshape(2, D)
    out_agg, out_ff = _tc_dense(neigh3, selfa, selff,
                                W_agg_v, W_ff_v, W_k, W_q, mu2)
    return (out_agg, out_ff)


# speedup vs baseline: 1.4088x; 1.0737x over previous
import functools

import jax
import jax.numpy as jnp
from jax import lax
from jax.experimental import pallas as pl
from jax.experimental.pallas import tpu as pltpu
from jax.experimental.pallas import tpu_sc as plsc

B = 10000
C = 4
M = 8
D = 128
NW = 32
CHUNK = 320
BLK = 8
STAGE_CHUNK = 80
SELF_CHUNK = 40
RES_RATE = 0.9


def _sc_gather_body(nodes_hbm, neigh_hbm, fa_hbm, ff_hbm,
                    out_n0, out_n1, out_n2, out_n3, out_a, out_f,
                    idx_v, idx2_v, nodes_v, rows0_v, rows1_v,
                    sums0_v, sums1_v, self_v,
                    gsem0, gsem1, wsem0, wsem1, ssem):
    rows_b = (rows0_v, rows1_v)
    sums_b = (sums0_v, sums1_v)
    out_ns = (out_n0, out_n1, out_n2, out_n3)
    wid = lax.axis_index("s") * 2 + lax.axis_index("c")
    start = pl.multiple_of(wid * CHUNK, 8)
    cnt = jnp.minimum(CHUNK, B - start)
    nstage = cnt // STAGE_CHUNK
    nself = cnt // SELF_CHUNK

    def stage_body(k, carry):
        off = pl.multiple_of(start + k * STAGE_CHUNK, 8)
        pltpu.sync_copy(neigh_hbm.at[pl.ds(off, STAGE_CHUNK)], idx_v)

        def repack(q, carry2):
            dst = k * (STAGE_CHUNK * C * M // 128) + q
            for j in range(8):
                idx2_v[dst, pl.ds(j * 16, 16)] = (
                    idx_v[4 * q + j // 2, pl.ds((j % 2) * 16, 16)])
            return carry2

        lax.fori_loop(0, STAGE_CHUNK * C * M // 128, repack, 0)
        return carry

    lax.fori_loop(0, nstage, stage_body, 0)

    def self_body(k, carry):
        noff = pl.multiple_of(start + k * SELF_CHUNK, 8)
        voff = pl.multiple_of(k * SELF_CHUNK, 8)
        pltpu.sync_copy(nodes_hbm.at[pl.ds(noff, SELF_CHUNK)],
                        nodes_v.at[pl.ds(voff, SELF_CHUNK)])
        idxs = nodes_v.at[pl.ds(voff, SELF_CHUNK)]
        ca = pltpu.async_copy(fa_hbm.at[idxs], self_v.at[0], ssem)
        cf = pltpu.async_copy(ff_hbm.at[idxs], self_v.at[1], ssem)
        ca.wait()
        cf.wait()
        pltpu.sync_copy(self_v.at[0], out_a.at[pl.ds(noff, SELF_CHUNK)])
        pltpu.sync_copy(self_v.at[1], out_f.at[pl.ds(noff, SELF_CHUNK)])
        return carry

    lax.fori_loop(0, nself, self_body, 0)

    nblk = cnt // BLK
    npair = nblk // 2

    gsems = (gsem0, gsem1)
    wsems = (wsem0, wsem1)

    def issue_gather(g, sb):
        pltpu.async_copy(fa_hbm.at[idx2_v.at[2 * g]],
                         rows_b[sb].at[pl.ds(0, 128)], gsems[sb])
        pltpu.async_copy(fa_hbm.at[idx2_v.at[2 * g + 1]],
                         rows_b[sb].at[pl.ds(128, 128)], gsems[sb])

    def wait_gather(sb):
        pltpu.make_async_copy(fa_hbm.at[idx2_v.at[0]],
                              rows_b[sb].at[pl.ds(0, 128)], gsems[sb]).wait()
        pltpu.make_async_copy(fa_hbm.at[idx2_v.at[0]],
                              rows_b[sb].at[pl.ds(128, 128)], gsems[sb]).wait()

    def compute(sb):
        @plsc.parallel_loop(0, C * BLK, 1, unroll=4)
        def obody(o):
            src = o * M
            for k in range(D // 16):
                r = [rows_b[sb][src + m, pl.ds(k * 16, 16)] for m in range(M)]
                s0 = r[0] + r[1]
                s1 = r[2] + r[3]
                s2 = r[4] + r[5]
                s3 = r[6] + r[7]
                sums_b[sb][o % C, o // C, pl.ds(k * 16, 16)] = (s0 + s1) + (s2 + s3)

    def issue_writes(g, sb):
        nbase = start + BLK * g
        for c in range(C):
            pltpu.async_copy(sums_b[sb].at[c], out_ns[c].at[pl.ds(nbase, BLK)],
                             wsems[sb])

    def wait_writes(sb):
        for c in range(C):
            pltpu.make_async_copy(sums_b[sb].at[c], out_ns[c].at[pl.ds(0, BLK)],
                                  wsems[sb]).wait()

    issue_gather(0, 0)

    def pair_body(p, carry):
        g0 = 2 * p
        issue_gather(g0 + 1, 1)
        wait_gather(0)

        @pl.when(p > 0)
        def _():
            wait_writes(0)

        compute(0)
        issue_writes(g0, 0)

        @pl.when(g0 + 2 < nblk)
        def _():
            issue_gather(g0 + 2, 0)

        wait_gather(1)

        @pl.when(p > 0)
        def _():
            wait_writes(1)

        compute(1)
        issue_writes(g0 + 1, 1)
        return carry

    lax.fori_loop(0, npair, pair_body, 0)
    wait_writes(0)
    wait_writes(1)


def _sc_gather(nodes, neigh2, feat_agg, feat_ff):
    mesh = plsc.VectorSubcoreMesh(core_axis_name="c", subcore_axis_name="s")
    f = functools.partial(
        pl.kernel,
        mesh=mesh,
        out_type=(
            jax.ShapeDtypeStruct((B, D), jnp.float32),
            jax.ShapeDtypeStruct((B, D), jnp.float32),
            jax.ShapeDtypeStruct((B, D), jnp.float32),
            jax.ShapeDtypeStruct((B, D), jnp.float32),
            jax.ShapeDtypeStruct((B, D), jnp.float32),
            jax.ShapeDtypeStruct((B, D), jnp.float32),
        ),
        scratch_types=[
            pltpu.VMEM((STAGE_CHUNK, C * M), jnp.int32),
            pltpu.VMEM((CHUNK * C * M // 128, 128), jnp.int32),
            pltpu.VMEM((CHUNK,), jnp.int32),
            pltpu.VMEM((2 * 128, D), jnp.float32),
            pltpu.VMEM((2 * 128, D), jnp.float32),
            pltpu.VMEM((C, BLK, D), jnp.float32),
            pltpu.VMEM((C, BLK, D), jnp.float32),
            pltpu.VMEM((2, SELF_CHUNK, D), jnp.float32),
            pltpu.SemaphoreType.DMA,
            pltpu.SemaphoreType.DMA,
            pltpu.SemaphoreType.DMA,
            pltpu.SemaphoreType.DMA,
            pltpu.SemaphoreType.DMA,
        ],
    )(_sc_gather_body)
    return f(nodes, neigh2, feat_agg, feat_ff)


def _tc_dense_body(n0_ref, n1_ref, n2_ref, n3_ref, a_ref, f_ref,
                   wagg_ref, wff_ref, wk_ref, wq_ref, mu_ref,
                   outa_ref, outf_ref):
    A = a_ref[...]
    F = f_ref[...]
    Wagg = wagg_ref[...]
    mu = mu_ref[...]
    mu_a = mu[0:1, :]
    mu_n = mu[1:2, :]

    dot = functools.partial(jnp.dot, preferred_element_type=jnp.float32)
    agg_v = dot(A, Wagg)
    ff_v = dot(F, wff_ref[...])
    Ka = dot(A, wk_ref[...])
    Kf = dot(F, wk_ref[...])
    Qa = dot(A, wq_ref[...])
    Qf = dot(F, wq_ref[...])

    a2 = jnp.sum(agg_v * agg_v, axis=1, keepdims=True)
    da = jnp.sum(agg_v * mu_a, axis=1, keepdims=True)

    neigh_aggs = []
    logits = []
    for ref in (n0_ref, n1_ref, n2_ref, n3_ref):
        NA = dot(ref[...] * (1.0 / M), Wagg)
        n2 = jnp.sum(NA * NA, axis=1, keepdims=True)
        dn = jnp.sum(NA * mu_n, axis=1, keepdims=True)
        norm = jnp.maximum(jnp.sqrt(a2 + n2), 1e-12)
        neigh_aggs.append(NA)
        logits.append((da + dn) / norm)

    mx = jnp.maximum(jnp.maximum(logits[0], logits[1]),
                     jnp.maximum(logits[2], logits[3]))
    es = [jnp.exp(l - mx) for l in logits]
    inv_z = 1.0 / (es[0] + es[1] + es[2] + es[3])
    comb = (es[0] * neigh_aggs[0] + es[1] * neigh_aggs[1]
            + es[2] * neigh_aggs[2] + es[3] * neigh_aggs[3]) * inv_z
    agg_v = (agg_v + comb) * 0.5

    inv_d = 1.0 / D
    s00 = jnp.sum(Ka * Qa, axis=1, keepdims=True) * inv_d
    s01 = jnp.sum(Ka * Qf, axis=1, keepdims=True) * inv_d
    s10 = jnp.sum(Kf * Qa, axis=1, keepdims=True) * inv_d
    s11 = jnp.sum(Kf * Qf, axis=1, keepdims=True) * inv_d

    m0 = jnp.maximum(s00, s01)
    e00 = jnp.exp(s00 - m0)
    e01 = jnp.exp(s01 - m0)
    iz0 = 1.0 / (e00 + e01)
    m1 = jnp.maximum(s10, s11)
    e10 = jnp.exp(s10 - m1)
    e11 = jnp.exp(s11 - m1)
    iz1 = 1.0 / (e10 + e11)

    new_a = (e00 * agg_v + e01 * ff_v) * iz0
    new_f = (e10 * agg_v + e11 * ff_v) * iz1

    xa = RES_RATE * agg_v + (1.0 - RES_RATE) * new_a
    xf = RES_RATE * ff_v + (1.0 - RES_RATE) * new_f
    outa_ref[...] = jnp.where(xa > 0, xa, jnp.exp(jnp.minimum(xa, 0.0)) - 1.0)
    outf_ref[...] = jnp.where(xf > 0, xf, jnp.exp(jnp.minimum(xf, 0.0)) - 1.0)


def _tc_dense(n0, n1, n2, n3, selfa, selff, W_agg_v, W_ff_v, W_k, W_q, mu2):
    R = 2000
    grid = (B // R,)
    nspec = [pl.BlockSpec((R, D), lambda i: (i, 0)) for _ in range(C)]
    rspec = pl.BlockSpec((R, D), lambda i: (i, 0))
    wspec = pl.BlockSpec((D, D), lambda i: (0, 0))
    muspec = pl.BlockSpec((2, D), lambda i: (0, 0))
    return pl.pallas_call(
        _tc_dense_body,
        grid=grid,
        in_specs=nspec + [rspec, rspec, wspec, wspec, wspec, wspec, muspec],
        out_specs=[rspec, rspec],
        out_shape=[jax.ShapeDtypeStruct((B, D), jnp.float32),
                   jax.ShapeDtypeStruct((B, D), jnp.float32)],
    )(n0, n1, n2, n3, selfa, selff,
      W_agg_v, W_ff_v, W_k, W_q, mu2)


def kernel(nodes, neigh_idx, feat_agg, feat_ff, W_agg_v, W_ff_v, W_k, W_q, W_mu):
    n0, n1, n2, n3, selfa, selff = _sc_gather(nodes, neigh_idx.reshape(B, C * M),
                                              feat_agg, feat_ff)
    mu2 = W_mu.reshape(2, D)
    out_agg, out_ff = _tc_dense(n0, n1, n2, n3, selfa, selff,
                                W_agg_v, W_ff_v, W_k, W_q, mu2)
    return (out_agg, out_ff)


# speedup vs baseline: 1.4705x; 1.0438x over previous
import functools

import jax
import jax.numpy as jnp
from jax import lax
from jax.experimental import pallas as pl
from jax.experimental.pallas import tpu as pltpu
from jax.experimental.pallas import tpu_sc as plsc

B = 10000
C = 4
M = 8
D = 128
NW = 32
CHUNK = 320
BLK = 8
STAGE_CHUNK = 80
SELF_CHUNK = 40
RES_RATE = 0.9


def _sc_gather_body(nodes_hbm, neigh_hbm, fa_hbm, ff_hbm,
                    out_n0, out_n1, out_n2, out_n3, out_a, out_f,
                    idx_v, idx2_v, nodes_v, rows0_v, rows1_v,
                    sums0_v, sums1_v, self_v, self2_v,
                    gsem0, gsem1, wsem0, wsem1, ssem, swsem):
    rows_b = (rows0_v, rows1_v)
    sums_b = (sums0_v, sums1_v)
    out_ns = (out_n0, out_n1, out_n2, out_n3)
    wid = lax.axis_index("s") * 2 + lax.axis_index("c")
    start = pl.multiple_of(wid * CHUNK, 8)
    cnt = jnp.minimum(CHUNK, B - start)
    nstage = cnt // STAGE_CHUNK
    nself = cnt // SELF_CHUNK

    def stage_body(k, carry):
        off = pl.multiple_of(start + k * STAGE_CHUNK, 8)
        voff = pl.multiple_of(k * STAGE_CHUNK, 8)
        pltpu.async_copy(nodes_hbm.at[pl.ds(off, STAGE_CHUNK)],
                         nodes_v.at[pl.ds(voff, STAGE_CHUNK)], ssem)
        pltpu.sync_copy(neigh_hbm.at[pl.ds(off, STAGE_CHUNK)], idx_v)

        def repack(q, carry2):
            dst = k * (STAGE_CHUNK * C * M // 128) + q
            for j in range(8):
                idx2_v[dst, pl.ds(j * 16, 16)] = (
                    idx_v[4 * q + j // 2, pl.ds((j % 2) * 16, 16)])
            return carry2

        lax.fori_loop(0, STAGE_CHUNK * C * M // 128, repack, 0)
        return carry

    lax.fori_loop(0, nstage, stage_body, 0)

    def stage_nodes_wait(k, carry):
        pltpu.make_async_copy(nodes_hbm.at[pl.ds(0, STAGE_CHUNK)],
                              nodes_v.at[pl.ds(0, STAGE_CHUNK)], ssem).wait()
        return carry

    lax.fori_loop(0, nstage, stage_nodes_wait, 0)

    nblk = cnt // BLK
    npair = nblk // 2
    nhalf = cnt // (2 * SELF_CHUNK)
    self_b = (self_v, self2_v)

    gsems = (gsem0, gsem1)
    wsems = (wsem0, wsem1)

    def issue_gather(g, sb):
        pltpu.async_copy(fa_hbm.at[idx2_v.at[2 * g]],
                         rows_b[sb].at[pl.ds(0, 128)], gsems[sb])
        pltpu.async_copy(fa_hbm.at[idx2_v.at[2 * g + 1]],
                         rows_b[sb].at[pl.ds(128, 128)], gsems[sb])

    def wait_gather(sb):
        pltpu.make_async_copy(fa_hbm.at[idx2_v.at[0]],
                              rows_b[sb].at[pl.ds(0, 128)], gsems[sb]).wait()
        pltpu.make_async_copy(fa_hbm.at[idx2_v.at[0]],
                              rows_b[sb].at[pl.ds(128, 128)], gsems[sb]).wait()

    def compute(sb):
        @plsc.parallel_loop(0, C * BLK, 1, unroll=4)
        def obody(o):
            src = o * M
            for k in range(D // 16):
                r = [rows_b[sb][src + m, pl.ds(k * 16, 16)] for m in range(M)]
                s0 = r[0] + r[1]
                s1 = r[2] + r[3]
                s2 = r[4] + r[5]
                s3 = r[6] + r[7]
                sums_b[sb][o % C, o // C, pl.ds(k * 16, 16)] = (s0 + s1) + (s2 + s3)

    def issue_writes(g, sb):
        nbase = start + BLK * g
        for c in range(C):
            pltpu.async_copy(sums_b[sb].at[c], out_ns[c].at[pl.ds(nbase, BLK)],
                             wsems[sb])

    def wait_writes(sb):
        for c in range(C):
            pltpu.make_async_copy(sums_b[sb].at[c], out_ns[c].at[pl.ds(0, BLK)],
                                  wsems[sb]).wait()

    def issue_self(p):
        for h in range(2):
            voff = 2 * p * SELF_CHUNK + h * SELF_CHUNK
            idxs = nodes_v.at[pl.ds(voff, SELF_CHUNK)]
            pltpu.async_copy(fa_hbm.at[idxs], self_b[h].at[0], ssem)
            pltpu.async_copy(ff_hbm.at[idxs], self_b[h].at[1], ssem)

    def wait_self_gathers():
        for _ in range(4):
            pltpu.make_async_copy(fa_hbm.at[nodes_v.at[pl.ds(0, SELF_CHUNK)]],
                                  self_v.at[0], ssem).wait()

    def issue_self_writes(p):
        for h in range(2):
            noff = start + 2 * p * SELF_CHUNK + h * SELF_CHUNK
            pltpu.async_copy(self_b[h].at[0], out_a.at[pl.ds(noff, SELF_CHUNK)],
                             swsem)
            pltpu.async_copy(self_b[h].at[1], out_f.at[pl.ds(noff, SELF_CHUNK)],
                             swsem)

    def wait_self_writes():
        for _ in range(4):
            pltpu.make_async_copy(self_v.at[0], out_a.at[pl.ds(0, SELF_CHUNK)],
                                  swsem).wait()

    issue_gather(0, 0)

    def pair_body(p, carry):
        g0 = 2 * p

        @pl.when(p < nhalf)
        def _():
            @pl.when(p > 0)
            def _():
                wait_self_writes()
            issue_self(p)

        issue_gather(g0 + 1, 1)
        wait_gather(0)

        @pl.when(p > 0)
        def _():
            wait_writes(0)

        compute(0)
        issue_writes(g0, 0)

        @pl.when(g0 + 2 < nblk)
        def _():
            issue_gather(g0 + 2, 0)

        wait_gather(1)

        @pl.when(p > 0)
        def _():
            wait_writes(1)

        compute(1)
        issue_writes(g0 + 1, 1)

        @pl.when(p < nhalf)
        def _():
            wait_self_gathers()
            issue_self_writes(p)
        return carry

    lax.fori_loop(0, npair, pair_body, 0)
    wait_writes(0)
    wait_writes(1)
    wait_self_writes()


def _sc_gather(nodes, neigh2, feat_agg, feat_ff):
    mesh = plsc.VectorSubcoreMesh(core_axis_name="c", subcore_axis_name="s")
    f = functools.partial(
        pl.kernel,
        mesh=mesh,
        out_type=(
            jax.ShapeDtypeStruct((B, D), jnp.float32),
            jax.ShapeDtypeStruct((B, D), jnp.float32),
            jax.ShapeDtypeStruct((B, D), jnp.float32),
            jax.ShapeDtypeStruct((B, D), jnp.float32),
            jax.ShapeDtypeStruct((B, D), jnp.float32),
            jax.ShapeDtypeStruct((B, D), jnp.float32),
        ),
        scratch_types=[
            pltpu.VMEM((STAGE_CHUNK, C * M), jnp.int32),
            pltpu.VMEM((CHUNK * C * M // 128, 128), jnp.int32),
            pltpu.VMEM((CHUNK,), jnp.int32),
            pltpu.VMEM((2 * 128, D), jnp.float32),
            pltpu.VMEM((2 * 128, D), jnp.float32),
            pltpu.VMEM((C, BLK, D), jnp.float32),
            pltpu.VMEM((C, BLK, D), jnp.float32),
            pltpu.VMEM((2, SELF_CHUNK, D), jnp.float32),
            pltpu.VMEM((2, SELF_CHUNK, D), jnp.float32),
            pltpu.SemaphoreType.DMA,
            pltpu.SemaphoreType.DMA,
            pltpu.SemaphoreType.DMA,
            pltpu.SemaphoreType.DMA,
            pltpu.SemaphoreType.DMA,
            pltpu.SemaphoreType.DMA,
        ],
    )(_sc_gather_body)
    return f(nodes, neigh2, feat_agg, feat_ff)


def _tc_dense_body(n0_ref, n1_ref, n2_ref, n3_ref, a_ref, f_ref,
                   wagg_ref, wff_ref, wk_ref, wq_ref, mu_ref,
                   outa_ref, outf_ref):
    A = a_ref[...]
    F = f_ref[...]
    Wagg = wagg_ref[...]
    mu = mu_ref[...]
    mu_a = mu[0:1, :]
    mu_n = mu[1:2, :]

    dot = functools.partial(jnp.dot, preferred_element_type=jnp.float32)
    agg_v = dot(A, Wagg)
    ff_v = dot(F, wff_ref[...])
    Ka = dot(A, wk_ref[...])
    Kf = dot(F, wk_ref[...])
    Qa = dot(A, wq_ref[...])
    Qf = dot(F, wq_ref[...])

    a2 = jnp.sum(agg_v * agg_v, axis=1, keepdims=True)
    da = jnp.sum(agg_v * mu_a, axis=1, keepdims=True)

    neigh_aggs = []
    logits = []
    for ref in (n0_ref, n1_ref, n2_ref, n3_ref):
        NA = dot(ref[...] * (1.0 / M), Wagg)
        n2 = jnp.sum(NA * NA, axis=1, keepdims=True)
        dn = jnp.sum(NA * mu_n, axis=1, keepdims=True)
        norm = jnp.maximum(jnp.sqrt(a2 + n2), 1e-12)
        neigh_aggs.append(NA)
        logits.append((da + dn) / norm)

    mx = jnp.maximum(jnp.maximum(logits[0], logits[1]),
                     jnp.maximum(logits[2], logits[3]))
    es = [jnp.exp(l - mx) for l in logits]
    inv_z = 1.0 / (es[0] + es[1] + es[2] + es[3])
    comb = (es[0] * neigh_aggs[0] + es[1] * neigh_aggs[1]
            + es[2] * neigh_aggs[2] + es[3] * neigh_aggs[3]) * inv_z
    agg_v = (agg_v + comb) * 0.5

    inv_d = 1.0 / D
    s00 = jnp.sum(Ka * Qa, axis=1, keepdims=True) * inv_d
    s01 = jnp.sum(Ka * Qf, axis=1, keepdims=True) * inv_d
    s10 = jnp.sum(Kf * Qa, axis=1, keepdims=True) * inv_d
    s11 = jnp.sum(Kf * Qf, axis=1, keepdims=True) * inv_d

    m0 = jnp.maximum(s00, s01)
    e00 = jnp.exp(s00 - m0)
    e01 = jnp.exp(s01 - m0)
    iz0 = 1.0 / (e00 + e01)
    m1 = jnp.maximum(s10, s11)
    e10 = jnp.exp(s10 - m1)
    e11 = jnp.exp(s11 - m1)
    iz1 = 1.0 / (e10 + e11)

    new_a = (e00 * agg_v + e01 * ff_v) * iz0
    new_f = (e10 * agg_v + e11 * ff_v) * iz1

    xa = RES_RATE * agg_v + (1.0 - RES_RATE) * new_a
    xf = RES_RATE * ff_v + (1.0 - RES_RATE) * new_f
    outa_ref[...] = jnp.where(xa > 0, xa, jnp.exp(jnp.minimum(xa, 0.0)) - 1.0)
    outf_ref[...] = jnp.where(xf > 0, xf, jnp.exp(jnp.minimum(xf, 0.0)) - 1.0)


def _tc_dense(n0, n1, n2, n3, selfa, selff, W_agg_v, W_ff_v, W_k, W_q, mu2):
    R = 2000
    grid = (B // R,)
    nspec = [pl.BlockSpec((R, D), lambda i: (i, 0)) for _ in range(C)]
    rspec = pl.BlockSpec((R, D), lambda i: (i, 0))
    wspec = pl.BlockSpec((D, D), lambda i: (0, 0))
    muspec = pl.BlockSpec((2, D), lambda i: (0, 0))
    return pl.pallas_call(
        _tc_dense_body,
        grid=grid,
        in_specs=nspec + [rspec, rspec, wspec, wspec, wspec, wspec, muspec],
        out_specs=[rspec, rspec],
        out_shape=[jax.ShapeDtypeStruct((B, D), jnp.float32),
                   jax.ShapeDtypeStruct((B, D), jnp.float32)],
    )(n0, n1, n2, n3, selfa, selff,
      W_agg_v, W_ff_v, W_k, W_q, mu2)


def kernel(nodes, neigh_idx, feat_agg, feat_ff, W_agg_v, W_ff_v, W_k, W_q, W_mu):
    n0, n1, n2, n3, selfa, selff = _sc_gather(nodes, neigh_idx.reshape(B, C * M),
                                              feat_agg, feat_ff)
    mu2 = W_mu.reshape(2, D)
    out_agg, out_ff = _tc_dense(n0, n1, n2, n3, selfa, selff,
                                W_agg_v, W_ff_v, W_k, W_q, mu2)
    return (out_agg, out_ff)


# speedup vs baseline: 1.4936x; 1.0157x over previous
import functools

import jax
import jax.numpy as jnp
from jax import lax
from jax.experimental import pallas as pl
from jax.experimental.pallas import tpu as pltpu
from jax.experimental.pallas import tpu_sc as plsc

B = 10000
C = 4
M = 8
D = 128
NW = 32
CHUNK = 320
BLK = 8
STAGE_CHUNK = 80
SELF_CHUNK = 40
RES_RATE = 0.9


def _sc_gather_body(nodes_hbm, neigh_hbm, fa_hbm, ff_hbm,
                    out_n0, out_n1, out_n2, out_n3, out_a, out_f,
                    idx_v, idx2_v, nodes_v, rows0_v, rows1_v,
                    sums0_v, sums1_v, self_v, self2_v,
                    gsem0, gsem1, wsem0, wsem1, ssem, swsem):
    rows_b = (rows0_v, rows1_v)
    sums_b = (sums0_v, sums1_v)
    out_ns = (out_n0, out_n1, out_n2, out_n3)
    wid = lax.axis_index("s") * 2 + lax.axis_index("c")
    start = pl.multiple_of(wid * CHUNK, 8)
    cnt = jnp.minimum(CHUNK, B - start)
    nstage = cnt // STAGE_CHUNK
    nself = cnt // SELF_CHUNK

    def stage_body(k, carry):
        off = pl.multiple_of(start + k * STAGE_CHUNK, 8)
        voff = pl.multiple_of(k * STAGE_CHUNK, 8)
        pltpu.async_copy(nodes_hbm.at[pl.ds(off, STAGE_CHUNK)],
                         nodes_v.at[pl.ds(voff, STAGE_CHUNK)], ssem)
        pltpu.sync_copy(neigh_hbm.at[pl.ds(off, STAGE_CHUNK)], idx_v)

        def repack(q, carry2):
            dst = k * (STAGE_CHUNK * C * M // 128) + q
            for j in range(8):
                idx2_v[dst, pl.ds(j * 16, 16)] = (
                    idx_v[4 * q + j // 2, pl.ds((j % 2) * 16, 16)])
            return carry2

        lax.fori_loop(0, STAGE_CHUNK * C * M // 128, repack, 0)
        return carry

    stage_body(0, 0)
    pltpu.async_copy(fa_hbm.at[idx2_v.at[0]],
                     rows0_v.at[pl.ds(0, 128)], gsem0)
    pltpu.async_copy(fa_hbm.at[idx2_v.at[1]],
                     rows0_v.at[pl.ds(128, 128)], gsem0)
    lax.fori_loop(1, nstage, stage_body, 0)

    def stage_nodes_wait(k, carry):
        pltpu.make_async_copy(nodes_hbm.at[pl.ds(0, STAGE_CHUNK)],
                              nodes_v.at[pl.ds(0, STAGE_CHUNK)], ssem).wait()
        return carry

    lax.fori_loop(0, nstage, stage_nodes_wait, 0)

    nblk = cnt // BLK
    npair = nblk // 2
    nhalf = cnt // (2 * SELF_CHUNK)
    self_b = (self_v, self2_v)

    gsems = (gsem0, gsem1)
    wsems = (wsem0, wsem1)

    def issue_gather(g, sb):
        pltpu.async_copy(fa_hbm.at[idx2_v.at[2 * g]],
                         rows_b[sb].at[pl.ds(0, 128)], gsems[sb])
        pltpu.async_copy(fa_hbm.at[idx2_v.at[2 * g + 1]],
                         rows_b[sb].at[pl.ds(128, 128)], gsems[sb])

    def wait_gather(sb):
        pltpu.make_async_copy(fa_hbm.at[idx2_v.at[0]],
                              rows_b[sb].at[pl.ds(0, 128)], gsems[sb]).wait()
        pltpu.make_async_copy(fa_hbm.at[idx2_v.at[0]],
                              rows_b[sb].at[pl.ds(128, 128)], gsems[sb]).wait()

    def compute(sb):
        @plsc.parallel_loop(0, C * BLK, 1, unroll=4)
        def obody(o):
            src = o * M
            for k in range(D // 16):
                r = [rows_b[sb][src + m, pl.ds(k * 16, 16)] for m in range(M)]
                s0 = r[0] + r[1]
                s1 = r[2] + r[3]
                s2 = r[4] + r[5]
                s3 = r[6] + r[7]
                sums_b[sb][o % C, o // C, pl.ds(k * 16, 16)] = (s0 + s1) + (s2 + s3)

    def issue_writes(g, sb):
        nbase = start + BLK * g
        for c in range(C):
            pltpu.async_copy(sums_b[sb].at[c], out_ns[c].at[pl.ds(nbase, BLK)],
                             wsems[sb])

    def wait_writes(sb):
        for c in range(C):
            pltpu.make_async_copy(sums_b[sb].at[c], out_ns[c].at[pl.ds(0, BLK)],
                                  wsems[sb]).wait()

    def issue_self(p):
        for h in range(2):
            voff = 2 * p * SELF_CHUNK + h * SELF_CHUNK
            idxs = nodes_v.at[pl.ds(voff, SELF_CHUNK)]
            pltpu.async_copy(fa_hbm.at[idxs], self_b[h].at[0], ssem)
            pltpu.async_copy(ff_hbm.at[idxs], self_b[h].at[1], ssem)

    def wait_self_gathers():
        for _ in range(4):
            pltpu.make_async_copy(fa_hbm.at[nodes_v.at[pl.ds(0, SELF_CHUNK)]],
                                  self_v.at[0], ssem).wait()

    def issue_self_writes(p):
        for h in range(2):
            noff = start + 2 * p * SELF_CHUNK + h * SELF_CHUNK
            pltpu.async_copy(self_b[h].at[0], out_a.at[pl.ds(noff, SELF_CHUNK)],
                             swsem)
            pltpu.async_copy(self_b[h].at[1], out_f.at[pl.ds(noff, SELF_CHUNK)],
                             swsem)

    def wait_self_writes():
        for _ in range(4):
            pltpu.make_async_copy(self_v.at[0], out_a.at[pl.ds(0, SELF_CHUNK)],
                                  swsem).wait()

    def pair_body(p, carry):
        g0 = 2 * p

        @pl.when(p < nhalf)
        def _():
            @pl.when(p > 0)
            def _():
                wait_self_writes()
            issue_self(p)

        issue_gather(g0 + 1, 1)
        wait_gather(0)

        @pl.when(p > 0)
        def _():
            wait_writes(0)

        compute(0)
        issue_writes(g0, 0)

        @pl.when(g0 + 2 < nblk)
        def _():
            issue_gather(g0 + 2, 0)

        wait_gather(1)

        @pl.when(p > 0)
        def _():
            wait_writes(1)

        compute(1)
        issue_writes(g0 + 1, 1)

        @pl.when(p < nhalf)
        def _():
            wait_self_gathers()
            issue_self_writes(p)
        return carry

    lax.fori_loop(0, npair, pair_body, 0)
    wait_writes(0)
    wait_writes(1)
    wait_self_writes()


def _sc_gather(nodes, neigh2, feat_agg, feat_ff):
    mesh = plsc.VectorSubcoreMesh(core_axis_name="c", subcore_axis_name="s")
    f = functools.partial(
        pl.kernel,
        mesh=mesh,
        out_type=(
            jax.ShapeDtypeStruct((B, D), jnp.float32),
            jax.ShapeDtypeStruct((B, D), jnp.float32),
            jax.ShapeDtypeStruct((B, D), jnp.float32),
            jax.ShapeDtypeStruct((B, D), jnp.float32),
            jax.ShapeDtypeStruct((B, D), jnp.float32),
            jax.ShapeDtypeStruct((B, D), jnp.float32),
        ),
        scratch_types=[
            pltpu.VMEM((STAGE_CHUNK, C * M), jnp.int32),
            pltpu.VMEM((CHUNK * C * M // 128, 128), jnp.int32),
            pltpu.VMEM((CHUNK,), jnp.int32),
            pltpu.VMEM((2 * 128, D), jnp.float32),
            pltpu.VMEM((2 * 128, D), jnp.float32),
            pltpu.VMEM((C, BLK, D), jnp.float32),
            pltpu.VMEM((C, BLK, D), jnp.float32),
            pltpu.VMEM((2, SELF_CHUNK, D), jnp.float32),
            pltpu.VMEM((2, SELF_CHUNK, D), jnp.float32),
            pltpu.SemaphoreType.DMA,
            pltpu.SemaphoreType.DMA,
            pltpu.SemaphoreType.DMA,
            pltpu.SemaphoreType.DMA,
            pltpu.SemaphoreType.DMA,
            pltpu.SemaphoreType.DMA,
        ],
    )(_sc_gather_body)
    return f(nodes, neigh2, feat_agg, feat_ff)


def _tc_dense_body(n0_ref, n1_ref, n2_ref, n3_ref, a_ref, f_ref,
                   wagg_ref, wff_ref, wk_ref, wq_ref, mu_ref,
                   outa_ref, outf_ref):
    A = a_ref[...]
    F = f_ref[...]
    Wagg = wagg_ref[...]
    mu = mu_ref[...]
    mu_a = mu[0:1, :]
    mu_n = mu[1:2, :]

    dot = functools.partial(jnp.dot, preferred_element_type=jnp.float32)
    agg_v = dot(A, Wagg)
    ff_v = dot(F, wff_ref[...])
    Ka = dot(A, wk_ref[...])
    Kf = dot(F, wk_ref[...])
    Qa = dot(A, wq_ref[...])
    Qf = dot(F, wq_ref[...])

    a2 = jnp.sum(agg_v * agg_v, axis=1, keepdims=True)
    da = jnp.sum(agg_v * mu_a, axis=1, keepdims=True)

    neigh_aggs = []
    logits = []
    for ref in (n0_ref, n1_ref, n2_ref, n3_ref):
        NA = dot(ref[...] * (1.0 / M), Wagg)
        n2 = jnp.sum(NA * NA, axis=1, keepdims=True)
        dn = jnp.sum(NA * mu_n, axis=1, keepdims=True)
        norm = jnp.maximum(jnp.sqrt(a2 + n2), 1e-12)
        neigh_aggs.append(NA)
        logits.append((da + dn) / norm)

    mx = jnp.maximum(jnp.maximum(logits[0], logits[1]),
                     jnp.maximum(logits[2], logits[3]))
    es = [jnp.exp(l - mx) for l in logits]
    inv_z = 1.0 / (es[0] + es[1] + es[2] + es[3])
    comb = (es[0] * neigh_aggs[0] + es[1] * neigh_aggs[1]
            + es[2] * neigh_aggs[2] + es[3] * neigh_aggs[3]) * inv_z
    agg_v = (agg_v + comb) * 0.5

    inv_d = 1.0 / D
    s00 = jnp.sum(Ka * Qa, axis=1, keepdims=True) * inv_d
    s01 = jnp.sum(Ka * Qf, axis=1, keepdims=True) * inv_d
    s10 = jnp.sum(Kf * Qa, axis=1, keepdims=True) * inv_d
    s11 = jnp.sum(Kf * Qf, axis=1, keepdims=True) * inv_d

    m0 = jnp.maximum(s00, s01)
    e00 = jnp.exp(s00 - m0)
    e01 = jnp.exp(s01 - m0)
    iz0 = 1.0 / (e00 + e01)
    m1 = jnp.maximum(s10, s11)
    e10 = jnp.exp(s10 - m1)
    e11 = jnp.exp(s11 - m1)
    iz1 = 1.0 / (e10 + e11)

    new_a = (e00 * agg_v + e01 * ff_v) * iz0
    new_f = (e10 * agg_v + e11 * ff_v) * iz1

    xa = RES_RATE * agg_v + (1.0 - RES_RATE) * new_a
    xf = RES_RATE * ff_v + (1.0 - RES_RATE) * new_f
    outa_ref[...] = jnp.where(xa > 0, xa, jnp.exp(jnp.minimum(xa, 0.0)) - 1.0)
    outf_ref[...] = jnp.where(xf > 0, xf, jnp.exp(jnp.minimum(xf, 0.0)) - 1.0)


def _tc_dense(n0, n1, n2, n3, selfa, selff, W_agg_v, W_ff_v, W_k, W_q, mu2):
    R = 2000
    grid = (B // R,)
    nspec = [pl.BlockSpec((R, D), lambda i: (i, 0)) for _ in range(C)]
    rspec = pl.BlockSpec((R, D), lambda i: (i, 0))
    wspec = pl.BlockSpec((D, D), lambda i: (0, 0))
    muspec = pl.BlockSpec((2, D), lambda i: (0, 0))
    return pl.pallas_call(
        _tc_dense_body,
        grid=grid,
        in_specs=nspec + [rspec, rspec, wspec, wspec, wspec, wspec, muspec],
        out_specs=[rspec, rspec],
        out_shape=[jax.ShapeDtypeStruct((B, D), jnp.float32),
                   jax.ShapeDtypeStruct((B, D), jnp.float32)],
    )(n0, n1, n2, n3, selfa, selff,
      W_agg_v, W_ff_v, W_k, W_q, mu2)


def kernel(nodes, neigh_idx, feat_agg, feat_ff, W_agg_v, W_ff_v, W_k, W_q, W_mu):
    n0, n1, n2, n3, selfa, selff = _sc_gather(nodes, neigh_idx.reshape(B, C * M),
                                              feat_agg, feat_ff)
    mu2 = W_mu.reshape(2, D)
    out_agg, out_ff = _tc_dense(n0, n1, n2, n3, selfa, selff,
                                W_agg_v, W_ff_v, W_k, W_q, mu2)
    return (out_agg, out_ff)
